# Initial kernel scaffold; baseline (speedup 1.0000x reference)
import jax
import jax.numpy as jnp
from jax.experimental import pallas as pl


def kernel(inputs, edge_index, W1_self, W1_neigh, b1, W2_self, W2_neigh, b2, Wm1, bm1, Wm2, bm2):
    raise NotImplementedError("write your pallas kernel here")


# speedup vs baseline: 4.2591x; 4.2591x over previous
import functools

import jax
import jax.numpy as jnp
from jax import lax
from jax.experimental import pallas as pl
from jax.experimental.pallas import tpu as pltpu
from jax.experimental.pallas import tpu_sc as plsc

N = 10000
E = 320000
D = 128

NC = 2
NS = 16
NW = NC * NS
EPW = E // NW
CH = 80
NCHUNK = EPW // CH
BLK = 80
NBLK = N // BLK


def _agg_body(mode, *refs):
    if mode == "feat":
        (h_hbm, src_hbm, dst_hbm, psum_hbm,
         acc, src_v, dst_v, rows_v, zbuf, sem) = refs
    else:
        (dst_hbm, psum_hbm,
         acc, dst_v, rows_v, zbuf, sem) = refs

    c = lax.axis_index("c")
    s = lax.axis_index("s")
    wid = s * NC + c

    zero16 = jnp.zeros((16,), jnp.float32)
    one16 = jnp.ones((16,), jnp.float32)

    def zb(i, _):
        zbuf[i // 8, pl.ds((i % 8) * 16, 16)] = zero16
        return 0
    lax.fori_loop(0, BLK * (D // 16), zb, 0)

    if mode == "deg":
        def ob(i, _):
            rows_v[i // 8, pl.ds((i % 8) * 16, 16)] = one16
            return 0
        lax.fori_loop(0, CH * (D // 16), ob, 0)

    def zs(k, _):
        b = k * NS + s

        @pl.when(b < NBLK)
        def _():
            pltpu.sync_copy(zbuf, acc.at[pl.ds(b * BLK, BLK)])
        return 0
    lax.fori_loop(0, (NBLK + NS - 1) // NS, zs, 0)
    plsc.subcore_barrier()

    def edge_chunk(i, _):
        base = wid * EPW + i * CH
        pltpu.sync_copy(dst_hbm.at[pl.ds(base, CH)], dst_v)
        if mode == "feat":
            pltpu.sync_copy(src_hbm.at[pl.ds(base, CH)], src_v)
            pltpu.async_copy(h_hbm.at[src_v], rows_v, sem).wait()
        pltpu.sync_copy(rows_v, acc.at[dst_v], add=True)
        return 0
    lax.fori_loop(0, NCHUNK, edge_chunk, 0)

    plsc.subcore_barrier()

    def wout(k, _):
        b = k * NS + s

        @pl.when(b < NBLK)
        def _():
            row0 = b * BLK
            pltpu.sync_copy(acc.at[pl.ds(row0, BLK)],
                            psum_hbm.at[c, pl.ds(row0, BLK)])
        return 0
    lax.fori_loop(0, (NBLK + NS - 1) // NS, wout, 0)


def _make_agg(mode):
    mesh = plsc.VectorSubcoreMesh(core_axis_name="c", subcore_axis_name="s")
    out_type = jax.ShapeDtypeStruct((NC, N, D), jnp.float32)
    scratch = [pltpu.VMEM_SHARED((N, D), jnp.float32)]
    if mode == "feat":
        scratch.append(pltpu.VMEM((CH,), jnp.int32))
    scratch += [
        pltpu.VMEM((CH,), jnp.int32),
        pltpu.VMEM((CH, D), jnp.float32),
        pltpu.VMEM((BLK, D), jnp.float32),
        pltpu.SemaphoreType.DMA,
    ]
    return pl.kernel(
        functools.partial(_agg_body, mode),
        out_type=(out_type,),
        mesh=mesh,
        scratch_types=scratch,
        name="sc_agg_" + mode,
    )


_agg_feat = _make_agg("feat")
_agg_deg = _make_agg("deg")


def _blk_recip(d0_ref, d1_ref):
    deg = d0_ref[...][:, 0] + d1_ref[...][:, 0]
    return 1.0 / jnp.clip(deg, 1.0, None)


def _tc1_body(x_ref, p0_ref, p1_ref, d0_ref, d1_ref, ws_ref, wn_ref, b_ref, out_ref):
    recip = _blk_recip(d0_ref, d1_ref)
    mean = (p0_ref[...] + p1_ref[...]) * recip[:, None]
    h = (jnp.dot(x_ref[...], ws_ref[...], preferred_element_type=jnp.float32)
         + jnp.dot(mean, wn_ref[...], preferred_element_type=jnp.float32)
         + b_ref[...])
    out_ref[...] = jnp.maximum(h, 0.0)


def _tc2_body(h_ref, q0_ref, q1_ref, d0_ref, d1_ref, ws_ref, wn_ref, b_ref,
              wm1_ref, bm1_ref, wm2_ref, bm2_ref, out_ref):
    recip = _blk_recip(d0_ref, d1_ref)
    mean = (q0_ref[...] + q1_ref[...]) * recip[:, None]
    h = (jnp.dot(h_ref[...], ws_ref[...], preferred_element_type=jnp.float32)
         + jnp.dot(mean, wn_ref[...], preferred_element_type=jnp.float32)
         + b_ref[...])
    h = jnp.maximum(h, 0.0)
    nrm = jnp.sqrt(jnp.sum(h * h, axis=1, keepdims=True))
    h = h / jnp.maximum(nrm, 1e-12)
    t = jnp.maximum(
        jnp.dot(h, wm1_ref[...], preferred_element_type=jnp.float32)
        + bm1_ref[...], 0.0)
    logits = (jnp.dot(t, wm2_ref[...], preferred_element_type=jnp.float32)
              + bm2_ref[...])
    m = jnp.max(logits, axis=1, keepdims=True)
    e = jnp.exp(logits - m)
    out_ref[...] = e / jnp.sum(e, axis=1, keepdims=True)


NB = 1000


def _tc1(x, p0, p1, d0, d1, ws, wn, b):
    grid = (N // NB,)
    return pl.pallas_call(
        _tc1_body,
        grid=grid,
        in_specs=[
            pl.BlockSpec((NB, D), lambda i: (i, 0)),
            pl.BlockSpec((NB, D), lambda i: (i, 0)),
            pl.BlockSpec((NB, D), lambda i: (i, 0)),
            pl.BlockSpec((NB, D), lambda i: (i, 0)),
            pl.BlockSpec((NB, D), lambda i: (i, 0)),
            pl.BlockSpec((D, D), lambda i: (0, 0)),
            pl.BlockSpec((D, D), lambda i: (0, 0)),
            pl.BlockSpec((1, D), lambda i: (0, 0)),
        ],
        out_specs=pl.BlockSpec((NB, D), lambda i: (i, 0)),
        out_shape=jax.ShapeDtypeStruct((N, D), jnp.float32),
    )(x, p0, p1, d0, d1, ws, wn, b)


def _tc2(h, q0, q1, d0, d1, ws, wn, b, wm1, bm1, wm2, bm2):
    grid = (N // NB,)
    return pl.pallas_call(
        _tc2_body,
        grid=grid,
        in_specs=[
            pl.BlockSpec((NB, D), lambda i: (i, 0)),
            pl.BlockSpec((NB, D), lambda i: (i, 0)),
            pl.BlockSpec((NB, D), lambda i: (i, 0)),
            pl.BlockSpec((NB, D), lambda i: (i, 0)),
            pl.BlockSpec((NB, D), lambda i: (i, 0)),
            pl.BlockSpec((D, D), lambda i: (0, 0)),
            pl.BlockSpec((D, D), lambda i: (0, 0)),
            pl.BlockSpec((1, D), lambda i: (0, 0)),
            pl.BlockSpec((D, 64), lambda i: (0, 0)),
            pl.BlockSpec((1, 64), lambda i: (0, 0)),
            pl.BlockSpec((64, 4), lambda i: (0, 0)),
            pl.BlockSpec((1, 4), lambda i: (0, 0)),
        ],
        out_specs=pl.BlockSpec((NB, 4), lambda i: (i, 0)),
        out_shape=jax.ShapeDtypeStruct((N, 4), jnp.float32),
    )(h, q0, q1, d0, d1, ws, wn, b, wm1, bm1, wm2, bm2)


@jax.jit
def kernel(inputs, edge_index, W1_self, W1_neigh, b1, W2_self, W2_neigh, b2,
           Wm1, bm1, Wm2, bm2):
    src = edge_index[0]
    dst = edge_index[1]

    (deg2,) = _agg_deg(dst)
    (psum1,) = _agg_feat(inputs, src, dst)
    h1 = _tc1(inputs, psum1[0], psum1[1], deg2[0], deg2[1],
              W1_self, W1_neigh, b1.reshape(1, D))
    (psum2,) = _agg_feat(h1, src, dst)
    out = _tc2(h1, psum2[0], psum2[1], deg2[0], deg2[1],
               W2_self, W2_neigh, b2.reshape(1, D),
               Wm1, bm1.reshape(1, 64), Wm2, bm2.reshape(1, 4))
    return out


# speedup vs baseline: 7.5755x; 1.7787x over previous
import functools

import jax
import jax.numpy as jnp
from jax import lax
from jax.experimental import pallas as pl
from jax.experimental.pallas import tpu as pltpu
from jax.experimental.pallas import tpu_sc as plsc

N = 10000
E = 320000
D = 128

NC = 2
NS = 16
NW = NC * NS
EPW = E // NW
CH = 80
NCHUNK = EPW // CH
BLK = 80
NBLK = N // BLK


def _agg_body(mode, *refs):
    if mode == "feat":
        (h_hbm, src_hbm, dst_hbm, psum_hbm,
         acc, srcs_v, dsts_v, rows0, rows1, zbuf,
         isem, gsem0, gsem1, ssem0, ssem1) = refs
        rows = (rows0, rows1)
        gsem = (gsem0, gsem1)
        ssem = (ssem0, ssem1)
    else:
        (dst_hbm, psum_hbm,
         acc, dsts_v, ones_v, zbuf, isem, ssem0) = refs

    c = lax.axis_index("c")
    s = lax.axis_index("s")
    wid = s * NC + c
    ebase = wid * EPW

    zero16 = jnp.zeros((16,), jnp.float32)
    one16 = jnp.ones((16,), jnp.float32)

    def load_dst(i, slot):
        pltpu.async_copy(dst_hbm.at[pl.ds(ebase + i * CH, CH)],
                         dsts_v.at[slot], isem)

    def wait_dst():
        pltpu.make_async_copy(dst_hbm.at[pl.ds(ebase, CH)],
                              dsts_v.at[0], isem).wait()

    if mode == "feat":
        def load_src(i, slot):
            pltpu.async_copy(src_hbm.at[pl.ds(ebase + i * CH, CH)],
                             srcs_v.at[slot], isem)

    def zb(i, _):
        zbuf[i // 8, pl.ds((i % 8) * 16, 16)] = zero16
        return 0
    lax.fori_loop(0, BLK * (D // 16), zb, 0)

    if mode == "deg":
        def ob(i, _):
            ones_v[i // 8, pl.ds((i % 8) * 16, 16)] = one16
            return 0
        lax.fori_loop(0, CH * (D // 16), ob, 0)

    def zs(k, _):
        b = k * NS + s

        @pl.when(b < NBLK)
        def _():
            pltpu.sync_copy(zbuf, acc.at[pl.ds(b * BLK, BLK)])
        return 0
    lax.fori_loop(0, (NBLK + NS - 1) // NS, zs, 0)
    plsc.subcore_barrier()

    if mode == "feat":
        load_src(0, 0)
        load_dst(0, 0)
        load_src(1, 1)
        load_dst(1, 1)
        wait_dst()
        wait_dst()
        pltpu.async_copy(h_hbm.at[srcs_v.at[0]], rows0, gsem0)

        def step(i, b2, b3):
            @pl.when(i + 1 < NCHUNK)
            def _():
                wait_dst()
                wait_dst()

            pltpu.make_async_copy(
                h_hbm.at[srcs_v.at[b3]], rows[b2], gsem[b2]).wait()

            @pl.when(i >= 1)
            def _():
                pltpu.make_async_copy(
                    rows[1 - b2], acc.at[dsts_v.at[(i - 1) % 3]],
                    ssem[1 - b2]).wait()

            @pl.when(i + 1 < NCHUNK)
            def _():
                pltpu.async_copy(
                    h_hbm.at[srcs_v.at[(b3 + 1) % 3]], rows[1 - b2],
                    gsem[1 - b2])

            pltpu.async_copy(
                rows[b2], acc.at[dsts_v.at[b3]], ssem[b2], add=True)

            @pl.when(i + 2 < NCHUNK)
            def _():
                load_src(i + 2, (b3 + 2) % 3)
                load_dst(i + 2, (b3 + 2) % 3)

        def macro(g, _):
            for u in range(6):
                i = 6 * g + u

                @pl.when(i < NCHUNK)
                def _():
                    step(i, u % 2, u % 3)
            return 0
        lax.fori_loop(0, (NCHUNK + 5) // 6, macro, 0)
        pltpu.make_async_copy(
            rows[(NCHUNK - 1) % 2], acc.at[dsts_v.at[(NCHUNK - 1) % 3]],
            ssem[(NCHUNK - 1) % 2]).wait()
    else:
        load_dst(0, 0)
        load_dst(1, 1)

        def dstep(i, b4):
            wait_dst()
            pltpu.async_copy(ones_v, acc.at[dsts_v.at[b4]], ssem0,
                             add=True)

            @pl.when(i >= 2)
            def _():
                pltpu.make_async_copy(
                    ones_v, acc.at[dsts_v.at[(i - 2) % 4]], ssem0).wait()

            @pl.when(i + 2 < NCHUNK)
            def _():
                load_dst(i + 2, (b4 + 2) % 4)

        def dmacro(g, _):
            for u in range(4):
                i = 4 * g + u

                @pl.when(i < NCHUNK)
                def _():
                    dstep(i, u)
            return 0
        lax.fori_loop(0, (NCHUNK + 3) // 4, dmacro, 0)
        for k in range(2):
            pltpu.make_async_copy(
                ones_v, acc.at[dsts_v.at[(NCHUNK - 2 + k) % 4]],
                ssem0).wait()

    plsc.subcore_barrier()

    def wout(k, _):
        b = k * NS + s

        @pl.when(b < NBLK)
        def _():
            row0 = b * BLK
            pltpu.sync_copy(acc.at[pl.ds(row0, BLK)],
                            psum_hbm.at[c, pl.ds(row0, BLK)])
        return 0
    lax.fori_loop(0, (NBLK + NS - 1) // NS, wout, 0)


def _make_agg(mode):
    mesh = plsc.VectorSubcoreMesh(core_axis_name="c", subcore_axis_name="s")
    out_type = jax.ShapeDtypeStruct((NC, N, D), jnp.float32)
    if mode == "feat":
        scratch = [
            pltpu.VMEM_SHARED((N, D), jnp.float32),
            pltpu.VMEM((3, CH), jnp.int32),
            pltpu.VMEM((3, CH), jnp.int32),
            pltpu.VMEM((CH, D), jnp.float32),
            pltpu.VMEM((CH, D), jnp.float32),
            pltpu.VMEM((BLK, D), jnp.float32),
            pltpu.SemaphoreType.DMA,
            pltpu.SemaphoreType.DMA,
            pltpu.SemaphoreType.DMA,
            pltpu.SemaphoreType.DMA,
            pltpu.SemaphoreType.DMA,
        ]
    else:
        scratch = [
            pltpu.VMEM_SHARED((N, D), jnp.float32),
            pltpu.VMEM((4, CH), jnp.int32),
            pltpu.VMEM((CH, D), jnp.float32),
            pltpu.VMEM((BLK, D), jnp.float32),
            pltpu.SemaphoreType.DMA,
            pltpu.SemaphoreType.DMA,
        ]
    return pl.kernel(
        functools.partial(_agg_body, mode),
        out_type=(out_type,),
        mesh=mesh,
        scratch_types=scratch,
        name="sc_agg_" + mode,
    )


_agg_feat = _make_agg("feat")
_agg_deg = _make_agg("deg")


def _blk_recip(d0_ref, d1_ref):
    deg = d0_ref[...][:, 0] + d1_ref[...][:, 0]
    return 1.0 / jnp.clip(deg, 1.0, None)


def _tc1_body(x_ref, p0_ref, p1_ref, d0_ref, d1_ref, ws_ref, wn_ref, b_ref, out_ref):
    recip = _blk_recip(d0_ref, d1_ref)
    mean = (p0_ref[...] + p1_ref[...]) * recip[:, None]
    h = (jnp.dot(x_ref[...], ws_ref[...], preferred_element_type=jnp.float32)
         + jnp.dot(mean, wn_ref[...], preferred_element_type=jnp.float32)
         + b_ref[...])
    out_ref[...] = jnp.maximum(h, 0.0)


def _tc2_body(h_ref, q0_ref, q1_ref, d0_ref, d1_ref, ws_ref, wn_ref, b_ref,
              wm1_ref, bm1_ref, wm2_ref, bm2_ref, out_ref):
    recip = _blk_recip(d0_ref, d1_ref)
    mean = (q0_ref[...] + q1_ref[...]) * recip[:, None]
    h = (jnp.dot(h_ref[...], ws_ref[...], preferred_element_type=jnp.float32)
         + jnp.dot(mean, wn_ref[...], preferred_element_type=jnp.float32)
         + b_ref[...])
    h = jnp.maximum(h, 0.0)
    nrm = jnp.sqrt(jnp.sum(h * h, axis=1, keepdims=True))
    h = h / jnp.maximum(nrm, 1e-12)
    t = jnp.maximum(
        jnp.dot(h, wm1_ref[...], preferred_element_type=jnp.float32)
        + bm1_ref[...], 0.0)
    logits = (jnp.dot(t, wm2_ref[...], preferred_element_type=jnp.float32)
              + bm2_ref[...])
    m = jnp.max(logits, axis=1, keepdims=True)
    e = jnp.exp(logits - m)
    out_ref[...] = e / jnp.sum(e, axis=1, keepdims=True)


NB = 1000


def _tc1(x, p0, p1, d0, d1, ws, wn, b):
    grid = (N // NB,)
    return pl.pallas_call(
        _tc1_body,
        grid=grid,
        in_specs=[
            pl.BlockSpec((NB, D), lambda i: (i, 0)),
            pl.BlockSpec((NB, D), lambda i: (i, 0)),
            pl.BlockSpec((NB, D), lambda i: (i, 0)),
            pl.BlockSpec((NB, D), lambda i: (i, 0)),
            pl.BlockSpec((NB, D), lambda i: (i, 0)),
            pl.BlockSpec((D, D), lambda i: (0, 0)),
            pl.BlockSpec((D, D), lambda i: (0, 0)),
            pl.BlockSpec((1, D), lambda i: (0, 0)),
        ],
        out_specs=pl.BlockSpec((NB, D), lambda i: (i, 0)),
        out_shape=jax.ShapeDtypeStruct((N, D), jnp.float32),
    )(x, p0, p1, d0, d1, ws, wn, b)


def _tc2(h, q0, q1, d0, d1, ws, wn, b, wm1, bm1, wm2, bm2):
    grid = (N // NB,)
    return pl.pallas_call(
        _tc2_body,
        grid=grid,
        in_specs=[
            pl.BlockSpec((NB, D), lambda i: (i, 0)),
            pl.BlockSpec((NB, D), lambda i: (i, 0)),
            pl.BlockSpec((NB, D), lambda i: (i, 0)),
            pl.BlockSpec((NB, D), lambda i: (i, 0)),
            pl.BlockSpec((NB, D), lambda i: (i, 0)),
            pl.BlockSpec((D, D), lambda i: (0, 0)),
            pl.BlockSpec((D, D), lambda i: (0, 0)),
            pl.BlockSpec((1, D), lambda i: (0, 0)),
            pl.BlockSpec((D, 64), lambda i: (0, 0)),
            pl.BlockSpec((1, 64), lambda i: (0, 0)),
            pl.BlockSpec((64, 4), lambda i: (0, 0)),
            pl.BlockSpec((1, 4), lambda i: (0, 0)),
        ],
        out_specs=pl.BlockSpec((NB, 4), lambda i: (i, 0)),
        out_shape=jax.ShapeDtypeStruct((N, 4), jnp.float32),
    )(h, q0, q1, d0, d1, ws, wn, b, wm1, bm1, wm2, bm2)


@jax.jit
def kernel(inputs, edge_index, W1_self, W1_neigh, b1, W2_self, W2_neigh, b2,
           Wm1, bm1, Wm2, bm2):
    src = edge_index[0]
    dst = edge_index[1]

    (deg2,) = _agg_deg(dst)
    (psum1,) = _agg_feat(inputs, src, dst)
    h1 = _tc1(inputs, psum1[0], psum1[1], deg2[0], deg2[1],
              W1_self, W1_neigh, b1.reshape(1, D))
    (psum2,) = _agg_feat(h1, src, dst)
    out = _tc2(h1, psum2[0], psum2[1], deg2[0], deg2[1],
               W2_self, W2_neigh, b2.reshape(1, D),
               Wm1, bm1.reshape(1, 64), Wm2, bm2.reshape(1, 4))
    return out


# speedup vs baseline: 7.5970x; 1.0028x over previous
import functools

import jax
import jax.numpy as jnp
from jax import lax
from jax.experimental import pallas as pl
from jax.experimental.pallas import tpu as pltpu
from jax.experimental.pallas import tpu_sc as plsc

N = 10000
E = 320000
D = 128

NC = 2
NS = 16
NW = NC * NS
EPW = E // NW
CH = 80
NCHUNK = EPW // CH
BLK = 80
NBLK = N // BLK


def _agg_body(mode, *refs):
    if mode == "feat":
        (h_hbm, src_hbm, dst_hbm, psum_hbm,
         acc, srcs_v, dsts_v, rows0, rows1, rows2, zbuf,
         isem, gsem0, gsem1, gsem2, ssem0, ssem1, ssem2, wsem) = refs
        rows = (rows0, rows1, rows2)
        gsem = (gsem0, gsem1, gsem2)
        ssem = (ssem0, ssem1, ssem2)
        IDX = 6
    else:
        (dst_hbm, psum_hbm,
         acc, dsts_v, ones_v, zbuf, isem, ssem0, wsem) = refs
        IDX = 4

    c = lax.axis_index("c")
    s = lax.axis_index("s")
    wid = s * NC + c
    ebase = wid * EPW

    zero16 = jnp.zeros((16,), jnp.float32)
    one16 = jnp.ones((16,), jnp.float32)

    def load_dst(i, slot):
        pltpu.async_copy(dst_hbm.at[pl.ds(ebase + i * CH, CH)],
                         dsts_v.at[slot], isem)

    def wait_idx():
        pltpu.make_async_copy(dst_hbm.at[pl.ds(ebase, CH)],
                              dsts_v.at[0], isem).wait()

    if mode == "feat":
        def load_src(i, slot):
            pltpu.async_copy(src_hbm.at[pl.ds(ebase + i * CH, CH)],
                             srcs_v.at[slot], isem)

        def load_idx(i, slot):
            load_src(i, slot)
            load_dst(i, slot)
    else:
        load_idx = load_dst

    for k in range(3 if mode == "feat" else 2):
        load_idx(k, k)

    def zb(i, _):
        zbuf[i // 8, pl.ds((i % 8) * 16, 16)] = zero16
        return 0
    lax.fori_loop(0, BLK * (D // 16), zb, 0)

    if mode == "deg":
        def ob(i, _):
            ones_v[i // 8, pl.ds((i % 8) * 16, 16)] = one16
            return 0
        lax.fori_loop(0, CH * (D // 16), ob, 0)

    def zs(k, _):
        b = k * NS + s

        @pl.when(b < NBLK)
        def _():
            pltpu.sync_copy(zbuf, acc.at[pl.ds(b * BLK, BLK)])
        return 0
    lax.fori_loop(0, (NBLK + NS - 1) // NS, zs, 0)
    plsc.subcore_barrier()

    if mode == "feat":
        wait_idx()
        wait_idx()
        pltpu.async_copy(h_hbm.at[srcs_v.at[0]], rows0, gsem0)
        wait_idx()
        wait_idx()

        def step(i, b3, b6):
            pltpu.make_async_copy(
                h_hbm.at[srcs_v.at[b6]], rows[b3], gsem[b3]).wait()

            @pl.when(i >= 2)
            def _():
                pltpu.make_async_copy(
                    rows[(b3 + 1) % 3], acc.at[dsts_v.at[(b6 + 4) % 6]],
                    ssem[(b3 + 1) % 3]).wait()

            @pl.when(i + 1 < NCHUNK)
            def _():
                pltpu.async_copy(
                    h_hbm.at[srcs_v.at[(b6 + 1) % 6]],
                    rows[(b3 + 1) % 3], gsem[(b3 + 1) % 3])

            pltpu.async_copy(
                rows[b3], acc.at[dsts_v.at[b6]], ssem[b3], add=True)

            @pl.when(i + 2 < NCHUNK)
            def _():
                wait_idx()
                wait_idx()

            @pl.when(i + 3 < NCHUNK)
            def _():
                load_idx(i + 3, (b6 + 3) % 6)

        def macro(g, _):
            for u in range(6):
                i = 6 * g + u

                @pl.when(i < NCHUNK)
                def _():
                    step(i, u % 3, u % 6)
            return 0
        lax.fori_loop(0, (NCHUNK + 5) // 6, macro, 0)
        for k in range(2):
            j = NCHUNK - 2 + k
            pltpu.make_async_copy(
                rows[j % 3], acc.at[dsts_v.at[j % 6]], ssem[j % 3]).wait()
    else:
        def dstep(i, b4):
            wait_idx()
            pltpu.async_copy(ones_v, acc.at[dsts_v.at[b4]], ssem0,
                             add=True)

            @pl.when(i >= 2)
            def _():
                pltpu.make_async_copy(
                    ones_v, acc.at[dsts_v.at[(i - 2) % 4]], ssem0).wait()

            @pl.when(i + 2 < NCHUNK)
            def _():
                load_dst(i + 2, (b4 + 2) % 4)

        def dmacro(g, _):
            for u in range(4):
                i = 4 * g + u

                @pl.when(i < NCHUNK)
                def _():
                    dstep(i, u)
            return 0
        lax.fori_loop(0, (NCHUNK + 3) // 4, dmacro, 0)
        for k in range(2):
            pltpu.make_async_copy(
                ones_v, acc.at[dsts_v.at[(NCHUNK - 2 + k) % 4]],
                ssem0).wait()

    plsc.subcore_barrier()

    def wout(k, _):
        b = k * NS + s

        @pl.when(b < NBLK)
        def _():
            row0 = b * BLK
            pltpu.async_copy(acc.at[pl.ds(row0, BLK)],
                             psum_hbm.at[c, pl.ds(row0, BLK)], wsem)

        @pl.when(k >= 2)
        def _():
            b2 = (k - 2) * NS + s

            @pl.when(b2 < NBLK)
            def _():
                pltpu.make_async_copy(
                    acc.at[pl.ds(0, BLK)],
                    psum_hbm.at[c, pl.ds(0, BLK)], wsem).wait()
        return 0
    NWO = (NBLK + NS - 1) // NS
    lax.fori_loop(0, NWO, wout, 0)
    for k in range(2):
        b2 = (NWO - 2 + k) * NS + s

        @pl.when(b2 < NBLK)
        def _():
            pltpu.make_async_copy(
                acc.at[pl.ds(0, BLK)],
                psum_hbm.at[c, pl.ds(0, BLK)], wsem).wait()


def _make_agg(mode):
    mesh = plsc.VectorSubcoreMesh(core_axis_name="c", subcore_axis_name="s")
    out_type = jax.ShapeDtypeStruct((NC, N, D), jnp.float32)
    if mode == "feat":
        scratch = [
            pltpu.VMEM_SHARED((N, D), jnp.float32),
            pltpu.VMEM((6, CH), jnp.int32),
            pltpu.VMEM((6, CH), jnp.int32),
            pltpu.VMEM((CH, D), jnp.float32),
            pltpu.VMEM((CH, D), jnp.float32),
            pltpu.VMEM((CH, D), jnp.float32),
            pltpu.VMEM((BLK, D), jnp.float32),
            pltpu.SemaphoreType.DMA,
            pltpu.SemaphoreType.DMA,
            pltpu.SemaphoreType.DMA,
            pltpu.SemaphoreType.DMA,
            pltpu.SemaphoreType.DMA,
            pltpu.SemaphoreType.DMA,
            pltpu.SemaphoreType.DMA,
            pltpu.SemaphoreType.DMA,
        ]
    else:
        scratch = [
            pltpu.VMEM_SHARED((N, D), jnp.float32),
            pltpu.VMEM((4, CH), jnp.int32),
            pltpu.VMEM((CH, D), jnp.float32),
            pltpu.VMEM((BLK, D), jnp.float32),
            pltpu.SemaphoreType.DMA,
            pltpu.SemaphoreType.DMA,
            pltpu.SemaphoreType.DMA,
        ]
    return pl.kernel(
        functools.partial(_agg_body, mode),
        out_type=(out_type,),
        mesh=mesh,
        scratch_types=scratch,
        name="sc_agg_" + mode,
    )


_agg_feat = _make_agg("feat")
_agg_deg = _make_agg("deg")


def _blk_recip(d0_ref, d1_ref):
    deg = d0_ref[...][:, 0] + d1_ref[...][:, 0]
    return 1.0 / jnp.clip(deg, 1.0, None)


def _tc1_body(x_ref, p0_ref, p1_ref, d0_ref, d1_ref, ws_ref, wn_ref, b_ref, out_ref):
    recip = _blk_recip(d0_ref, d1_ref)
    mean = (p0_ref[...] + p1_ref[...]) * recip[:, None]
    h = (jnp.dot(x_ref[...], ws_ref[...], preferred_element_type=jnp.float32)
         + jnp.dot(mean, wn_ref[...], preferred_element_type=jnp.float32)
         + b_ref[...])
    out_ref[...] = jnp.maximum(h, 0.0)


def _tc2_body(h_ref, q0_ref, q1_ref, d0_ref, d1_ref, ws_ref, wn_ref, b_ref,
              wm1_ref, bm1_ref, wm2_ref, bm2_ref, out_ref):
    recip = _blk_recip(d0_ref, d1_ref)
    mean = (q0_ref[...] + q1_ref[...]) * recip[:, None]
    h = (jnp.dot(h_ref[...], ws_ref[...], preferred_element_type=jnp.float32)
         + jnp.dot(mean, wn_ref[...], preferred_element_type=jnp.float32)
         + b_ref[...])
    h = jnp.maximum(h, 0.0)
    nrm = jnp.sqrt(jnp.sum(h * h, axis=1, keepdims=True))
    h = h / jnp.maximum(nrm, 1e-12)
    t = jnp.maximum(
        jnp.dot(h, wm1_ref[...], preferred_element_type=jnp.float32)
        + bm1_ref[...], 0.0)
    logits = (jnp.dot(t, wm2_ref[...], preferred_element_type=jnp.float32)
              + bm2_ref[...])
    m = jnp.max(logits, axis=1, keepdims=True)
    e = jnp.exp(logits - m)
    out_ref[...] = e / jnp.sum(e, axis=1, keepdims=True)


NB = 1000


def _tc1(x, p0, p1, d0, d1, ws, wn, b):
    grid = (N // NB,)
    return pl.pallas_call(
        _tc1_body,
        grid=grid,
        in_specs=[
            pl.BlockSpec((NB, D), lambda i: (i, 0)),
            pl.BlockSpec((NB, D), lambda i: (i, 0)),
            pl.BlockSpec((NB, D), lambda i: (i, 0)),
            pl.BlockSpec((NB, D), lambda i: (i, 0)),
            pl.BlockSpec((NB, D), lambda i: (i, 0)),
            pl.BlockSpec((D, D), lambda i: (0, 0)),
            pl.BlockSpec((D, D), lambda i: (0, 0)),
            pl.BlockSpec((1, D), lambda i: (0, 0)),
        ],
        out_specs=pl.BlockSpec((NB, D), lambda i: (i, 0)),
        out_shape=jax.ShapeDtypeStruct((N, D), jnp.float32),
    )(x, p0, p1, d0, d1, ws, wn, b)


def _tc2(h, q0, q1, d0, d1, ws, wn, b, wm1, bm1, wm2, bm2):
    grid = (N // NB,)
    return pl.pallas_call(
        _tc2_body,
        grid=grid,
        in_specs=[
            pl.BlockSpec((NB, D), lambda i: (i, 0)),
            pl.BlockSpec((NB, D), lambda i: (i, 0)),
            pl.BlockSpec((NB, D), lambda i: (i, 0)),
            pl.BlockSpec((NB, D), lambda i: (i, 0)),
            pl.BlockSpec((NB, D), lambda i: (i, 0)),
            pl.BlockSpec((D, D), lambda i: (0, 0)),
            pl.BlockSpec((D, D), lambda i: (0, 0)),
            pl.BlockSpec((1, D), lambda i: (0, 0)),
            pl.BlockSpec((D, 64), lambda i: (0, 0)),
            pl.BlockSpec((1, 64), lambda i: (0, 0)),
            pl.BlockSpec((64, 4), lambda i: (0, 0)),
            pl.BlockSpec((1, 4), lambda i: (0, 0)),
        ],
        out_specs=pl.BlockSpec((NB, 4), lambda i: (i, 0)),
        out_shape=jax.ShapeDtypeStruct((N, 4), jnp.float32),
    )(h, q0, q1, d0, d1, ws, wn, b, wm1, bm1, wm2, bm2)


@jax.jit
def kernel(inputs, edge_index, W1_self, W1_neigh, b1, W2_self, W2_neigh, b2,
           Wm1, bm1, Wm2, bm2):
    src = edge_index[0]
    dst = edge_index[1]

    (deg2,) = _agg_deg(dst)
    (psum1,) = _agg_feat(inputs, src, dst)
    h1 = _tc1(inputs, psum1[0], psum1[1], deg2[0], deg2[1],
              W1_self, W1_neigh, b1.reshape(1, D))
    (psum2,) = _agg_feat(h1, src, dst)
    out = _tc2(h1, psum2[0], psum2[1], deg2[0], deg2[1],
               W2_self, W2_neigh, b2.reshape(1, D),
               Wm1, bm1.reshape(1, 64), Wm2, bm2.reshape(1, 4))
    return out


# speedup vs baseline: 8.0326x; 1.0573x over previous
import functools

import jax
import jax.numpy as jnp
from jax import lax
from jax.experimental import pallas as pl
from jax.experimental.pallas import tpu as pltpu
from jax.experimental.pallas import tpu_sc as plsc

N = 10000
E = 320000
D = 128

NC = 2
NS = 16
NW = NC * NS
EPW = E // NW
CH = 80
NCHUNK = EPW // CH
BLK = 80
NBLK = N // BLK


def _agg_body(with_deg, *refs):
    if with_deg:
        (h_hbm, src_hbm, dst_hbm, psum_hbm, deg_hbm,
         acc, srcs_v, dsts_v, rows0, rows1, rows2, zbuf,
         isem, gsem0, gsem1, gsem2, ssem0, ssem1, ssem2, wsem) = refs
    else:
        (h_hbm, src_hbm, dst_hbm, psum_hbm,
         acc, srcs_v, dsts_v, rows0, rows1, rows2, zbuf,
         isem, gsem0, gsem1, gsem2, ssem0, ssem1, ssem2, wsem) = refs
    rows = (rows0, rows1, rows2)
    gsem = (gsem0, gsem1, gsem2)
    ssem = (ssem0, ssem1, ssem2)

    c = lax.axis_index("c")
    s = lax.axis_index("s")
    wid = s * NC + c
    ebase = wid * EPW

    zero16 = jnp.zeros((16,), jnp.float32)
    one16 = jnp.ones((16,), jnp.float32)

    def load_dst(i, slot):
        pltpu.async_copy(dst_hbm.at[pl.ds(ebase + i * CH, CH)],
                         dsts_v.at[slot], isem)

    def load_src(i, slot):
        pltpu.async_copy(src_hbm.at[pl.ds(ebase + i * CH, CH)],
                         srcs_v.at[slot], isem)

    def wait_idx():
        pltpu.make_async_copy(dst_hbm.at[pl.ds(ebase, CH)],
                              dsts_v.at[0], isem).wait()

    def load_idx(i, slot):
        load_src(i, slot)
        load_dst(i, slot)

    for k in range(3):
        load_idx(k, k)

    def zb(i, _):
        zbuf[i // 8, pl.ds((i % 8) * 16, 16)] = zero16
        return 0
    lax.fori_loop(0, BLK * (D // 16), zb, 0)

    def zero_acc(k, _):
        b = k * NS + s

        @pl.when(b < NBLK)
        def _():
            pltpu.sync_copy(zbuf, acc.at[pl.ds(b * BLK, BLK)])
        return 0
    lax.fori_loop(0, (NBLK + NS - 1) // NS, zero_acc, 0)
    plsc.subcore_barrier()

    wait_idx()
    wait_idx()
    pltpu.async_copy(h_hbm.at[srcs_v.at[0]], rows0, gsem0)
    wait_idx()
    wait_idx()

    def step(i, b3, b6):
        pltpu.make_async_copy(
            h_hbm.at[srcs_v.at[b6]], rows[b3], gsem[b3]).wait()

        @pl.when(i >= 2)
        def _():
            pltpu.make_async_copy(
                rows[(b3 + 1) % 3], acc.at[dsts_v.at[(b6 + 4) % 6]],
                ssem[(b3 + 1) % 3]).wait()

        @pl.when(i + 1 < NCHUNK)
        def _():
            pltpu.async_copy(
                h_hbm.at[srcs_v.at[(b6 + 1) % 6]],
                rows[(b3 + 1) % 3], gsem[(b3 + 1) % 3])

        pltpu.async_copy(
            rows[b3], acc.at[dsts_v.at[b6]], ssem[b3], add=True)

        @pl.when(i + 2 < NCHUNK)
        def _():
            wait_idx()
            wait_idx()

        @pl.when(i + 3 < NCHUNK)
        def _():
            load_idx(i + 3, (b6 + 3) % 6)

    def macro(g, _):
        for u in range(6):
            i = 6 * g + u

            @pl.when(i < NCHUNK)
            def _():
                step(i, u % 3, u % 6)
        return 0
    lax.fori_loop(0, (NCHUNK + 5) // 6, macro, 0)
    for k in range(2):
        j = NCHUNK - 2 + k
        pltpu.make_async_copy(
            rows[j % 3], acc.at[dsts_v.at[j % 6]], ssem[j % 3]).wait()

    plsc.subcore_barrier()

    def writeout(out_hbm):
        NWO = (NBLK + NS - 1) // NS

        def wout(k, _):
            b = k * NS + s

            @pl.when(b < NBLK)
            def _():
                row0 = b * BLK
                pltpu.async_copy(acc.at[pl.ds(row0, BLK)],
                                 out_hbm.at[c, pl.ds(row0, BLK)], wsem)

            @pl.when(k >= 2)
            def _():
                b2 = (k - 2) * NS + s

                @pl.when(b2 < NBLK)
                def _():
                    pltpu.make_async_copy(
                        acc.at[pl.ds(0, BLK)],
                        out_hbm.at[c, pl.ds(0, BLK)], wsem).wait()
            return 0
        lax.fori_loop(0, NWO, wout, 0)
        for k in range(2):
            b2 = (NWO - 2 + k) * NS + s

            @pl.when(b2 < NBLK)
            def _():
                pltpu.make_async_copy(
                    acc.at[pl.ds(0, BLK)],
                    out_hbm.at[c, pl.ds(0, BLK)], wsem).wait()

    writeout(psum_hbm)

    if with_deg:
        plsc.subcore_barrier()
        load_dst(0, 0)
        load_dst(1, 1)

        def of(i, _):
            rows0[i // 8, pl.ds((i % 8) * 16, 16)] = one16
            return 0
        lax.fori_loop(0, CH * (D // 16), of, 0)

        lax.fori_loop(0, (NBLK + NS - 1) // NS, zero_acc, 0)
        plsc.subcore_barrier()

        def dstep(i, b4):
            wait_idx()
            pltpu.async_copy(rows0, acc.at[dsts_v.at[b4]], ssem0,
                             add=True)

            @pl.when(i >= 2)
            def _():
                pltpu.make_async_copy(
                    rows0, acc.at[dsts_v.at[(i - 2) % 4]], ssem0).wait()

            @pl.when(i + 2 < NCHUNK)
            def _():
                load_dst(i + 2, (b4 + 2) % 4)

        def dmacro(g, _):
            for u in range(4):
                i = 4 * g + u

                @pl.when(i < NCHUNK)
                def _():
                    dstep(i, u)
            return 0
        lax.fori_loop(0, (NCHUNK + 3) // 4, dmacro, 0)
        for k in range(2):
            pltpu.make_async_copy(
                rows0, acc.at[dsts_v.at[(NCHUNK - 2 + k) % 4]],
                ssem0).wait()
        plsc.subcore_barrier()
        writeout(deg_hbm)


def _make_agg(with_deg):
    mesh = plsc.VectorSubcoreMesh(core_axis_name="c", subcore_axis_name="s")
    out_type = [jax.ShapeDtypeStruct((NC, N, D), jnp.float32)]
    if with_deg:
        out_type.append(jax.ShapeDtypeStruct((NC, N, D), jnp.float32))
    scratch = [
        pltpu.VMEM_SHARED((N, D), jnp.float32),
        pltpu.VMEM((6, CH), jnp.int32),
        pltpu.VMEM((6, CH), jnp.int32),
        pltpu.VMEM((CH, D), jnp.float32),
        pltpu.VMEM((CH, D), jnp.float32),
        pltpu.VMEM((CH, D), jnp.float32),
        pltpu.VMEM((BLK, D), jnp.float32),
        pltpu.SemaphoreType.DMA,
        pltpu.SemaphoreType.DMA,
        pltpu.SemaphoreType.DMA,
        pltpu.SemaphoreType.DMA,
        pltpu.SemaphoreType.DMA,
        pltpu.SemaphoreType.DMA,
        pltpu.SemaphoreType.DMA,
        pltpu.SemaphoreType.DMA,
    ]
    return pl.kernel(
        functools.partial(_agg_body, with_deg),
        out_type=tuple(out_type),
        mesh=mesh,
        scratch_types=scratch,
        name="sc_agg_featdeg" if with_deg else "sc_agg_feat",
    )


_agg_featdeg = _make_agg(True)
_agg_feat = _make_agg(False)


def _blk_recip(d_ref):
    d = d_ref[...]
    deg = d[0, :, 0] + d[1, :, 0]
    return 1.0 / jnp.clip(deg, 1.0, None)


def _tc1_body(x_ref, p_ref, d_ref, ws_ref, wn_ref, b_ref, out_ref):
    recip = _blk_recip(d_ref)
    p = p_ref[...]
    mean = (p[0] + p[1]) * recip[:, None]
    h = (jnp.dot(x_ref[...], ws_ref[...], preferred_element_type=jnp.float32)
         + jnp.dot(mean, wn_ref[...], preferred_element_type=jnp.float32)
         + b_ref[...])
    out_ref[...] = jnp.maximum(h, 0.0)


def _tc2_body(h_ref, q_ref, d_ref, ws_ref, wn_ref, b_ref,
              wm1_ref, bm1_ref, wm2_ref, bm2_ref, out_ref):
    recip = _blk_recip(d_ref)
    q = q_ref[...]
    mean = (q[0] + q[1]) * recip[:, None]
    h = (jnp.dot(h_ref[...], ws_ref[...], preferred_element_type=jnp.float32)
         + jnp.dot(mean, wn_ref[...], preferred_element_type=jnp.float32)
         + b_ref[...])
    h = jnp.maximum(h, 0.0)
    nrm = jnp.sqrt(jnp.sum(h * h, axis=1, keepdims=True))
    h = h / jnp.maximum(nrm, 1e-12)
    t = jnp.maximum(
        jnp.dot(h, wm1_ref[...], preferred_element_type=jnp.float32)
        + bm1_ref[...], 0.0)
    logits = (jnp.dot(t, wm2_ref[...], preferred_element_type=jnp.float32)
              + bm2_ref[...])
    m = jnp.max(logits, axis=1, keepdims=True)
    e = jnp.exp(logits - m)
    out_ref[...] = e / jnp.sum(e, axis=1, keepdims=True)


NB = 1000


def _tc1(x, p, d, ws, wn, b):
    grid = (N // NB,)
    return pl.pallas_call(
        _tc1_body,
        grid=grid,
        in_specs=[
            pl.BlockSpec((NB, D), lambda i: (i, 0)),
            pl.BlockSpec((NC, NB, D), lambda i: (0, i, 0)),
            pl.BlockSpec((NC, NB, D), lambda i: (0, i, 0)),
            pl.BlockSpec((D, D), lambda i: (0, 0)),
            pl.BlockSpec((D, D), lambda i: (0, 0)),
            pl.BlockSpec((1, D), lambda i: (0, 0)),
        ],
        out_specs=pl.BlockSpec((NB, D), lambda i: (i, 0)),
        out_shape=jax.ShapeDtypeStruct((N, D), jnp.float32),
    )(x, p, d, ws, wn, b)


def _tc2(h, q, d, ws, wn, b, wm1, bm1, wm2, bm2):
    grid = (N // NB,)
    return pl.pallas_call(
        _tc2_body,
        grid=grid,
        in_specs=[
            pl.BlockSpec((NB, D), lambda i: (i, 0)),
            pl.BlockSpec((NC, NB, D), lambda i: (0, i, 0)),
            pl.BlockSpec((NC, NB, D), lambda i: (0, i, 0)),
            pl.BlockSpec((D, D), lambda i: (0, 0)),
            pl.BlockSpec((D, D), lambda i: (0, 0)),
            pl.BlockSpec((1, D), lambda i: (0, 0)),
            pl.BlockSpec((D, 64), lambda i: (0, 0)),
            pl.BlockSpec((1, 64), lambda i: (0, 0)),
            pl.BlockSpec((64, 4), lambda i: (0, 0)),
            pl.BlockSpec((1, 4), lambda i: (0, 0)),
        ],
        out_specs=pl.BlockSpec((NB, 4), lambda i: (i, 0)),
        out_shape=jax.ShapeDtypeStruct((N, 4), jnp.float32),
    )(h, q, d, ws, wn, b, wm1, bm1, wm2, bm2)


@jax.jit
def kernel(inputs, edge_index, W1_self, W1_neigh, b1, W2_self, W2_neigh, b2,
           Wm1, bm1, Wm2, bm2):
    src = edge_index[0]
    dst = edge_index[1]

    psum1, deg2 = _agg_featdeg(inputs, src, dst)
    h1 = _tc1(inputs, psum1, deg2, W1_self, W1_neigh, b1.reshape(1, D))
    (psum2,) = _agg_feat(h1, src, dst)
    out = _tc2(h1, psum2, deg2,
               W2_self, W2_neigh, b2.reshape(1, D),
               Wm1, bm1.reshape(1, 64), Wm2, bm2.reshape(1, 4))
    return out


# speedup vs baseline: 8.0977x; 1.0081x over previous
import functools

import jax
import jax.numpy as jnp
from jax import lax
from jax.experimental import pallas as pl
from jax.experimental.pallas import tpu as pltpu
from jax.experimental.pallas import tpu_sc as plsc

N = 10000
E = 320000
D = 128

NC = 2
NS = 16
NW = NC * NS
EPW = E // NW
CH = 80
NCHUNK = EPW // CH
BLK = 80
NBLK = N // BLK


def _agg_body(with_deg, *refs):
    if with_deg:
        (h_hbm, src_hbm, dst_hbm, psum_hbm, deg_hbm,
         acc, srcs_v, dsts_v, rows0, rows1, rows2, zbuf,
         isem, gsem0, gsem1, gsem2, ssem0, ssem1, ssem2, wsem) = refs
    else:
        (h_hbm, src_hbm, dst_hbm, psum_hbm,
         acc, srcs_v, dsts_v, rows0, rows1, rows2, zbuf,
         isem, gsem0, gsem1, gsem2, ssem0, ssem1, ssem2, wsem) = refs
    rows = (rows0, rows1, rows2)
    gsem = (gsem0, gsem1, gsem2)
    ssem = (ssem0, ssem1, ssem2)

    c = lax.axis_index("c")
    s = lax.axis_index("s")
    wid = s * NC + c
    ebase = wid * EPW

    zero16 = jnp.zeros((16,), jnp.float32)
    one16 = jnp.ones((16,), jnp.float32)

    def load_dst(i, slot):
        pltpu.async_copy(dst_hbm.at[pl.ds(ebase + i * CH, CH)],
                         dsts_v.at[slot], isem)

    def load_src(i, slot):
        pltpu.async_copy(src_hbm.at[pl.ds(ebase + i * CH, CH)],
                         srcs_v.at[slot], isem)

    def wait_idx():
        pltpu.make_async_copy(dst_hbm.at[pl.ds(ebase, CH)],
                              dsts_v.at[0], isem).wait()

    def load_idx(i, slot):
        load_src(i, slot)
        load_dst(i, slot)

    for k in range(3):
        load_idx(k, k)

    def zb(i, _):
        zbuf[i // 8, pl.ds((i % 8) * 16, 16)] = zero16
        return 0
    lax.fori_loop(0, BLK * (D // 16), zb, 0)

    def zero_acc(k, _):
        b = k * NS + s

        @pl.when(b < NBLK)
        def _():
            pltpu.sync_copy(zbuf, acc.at[pl.ds(b * BLK, BLK)])
        return 0
    lax.fori_loop(0, (NBLK + NS - 1) // NS, zero_acc, 0)
    plsc.subcore_barrier()

    wait_idx()
    wait_idx()
    pltpu.async_copy(h_hbm.at[srcs_v.at[0]], rows0, gsem0)
    wait_idx()
    wait_idx()

    def step(i, b3, b6):
        pltpu.make_async_copy(
            h_hbm.at[srcs_v.at[b6]], rows[b3], gsem[b3]).wait()

        @pl.when(i >= 2)
        def _():
            pltpu.make_async_copy(
                rows[(b3 + 1) % 3], acc.at[dsts_v.at[(b6 + 4) % 6]],
                ssem[(b3 + 1) % 3]).wait()

        @pl.when(i + 1 < NCHUNK)
        def _():
            pltpu.async_copy(
                h_hbm.at[srcs_v.at[(b6 + 1) % 6]],
                rows[(b3 + 1) % 3], gsem[(b3 + 1) % 3])

        pltpu.async_copy(
            rows[b3], acc.at[dsts_v.at[b6]], ssem[b3], add=True)

        @pl.when(i + 2 < NCHUNK)
        def _():
            wait_idx()
            wait_idx()

        @pl.when(i + 3 < NCHUNK)
        def _():
            load_idx(i + 3, (b6 + 3) % 6)

    def macro(g, _):
        for u in range(6):
            i = 6 * g + u

            @pl.when(i < NCHUNK)
            def _():
                step(i, u % 3, u % 6)
        return 0
    lax.fori_loop(0, (NCHUNK + 5) // 6, macro, 0)
    for k in range(2):
        j = NCHUNK - 2 + k
        pltpu.make_async_copy(
            rows[j % 3], acc.at[dsts_v.at[j % 6]], ssem[j % 3]).wait()

    plsc.subcore_barrier()

    def writeout(out_hbm):
        NWO = (NBLK + NS - 1) // NS

        def wout(k, _):
            b = k * NS + s

            @pl.when(b < NBLK)
            def _():
                row0 = b * BLK
                pltpu.async_copy(acc.at[pl.ds(row0, BLK)],
                                 out_hbm.at[c, pl.ds(row0, BLK)], wsem)

            @pl.when(k >= 2)
            def _():
                b2 = (k - 2) * NS + s

                @pl.when(b2 < NBLK)
                def _():
                    pltpu.make_async_copy(
                        acc.at[pl.ds(0, BLK)],
                        out_hbm.at[c, pl.ds(0, BLK)], wsem).wait()
            return 0
        lax.fori_loop(0, NWO, wout, 0)
        for k in range(2):
            b2 = (NWO - 2 + k) * NS + s

            @pl.when(b2 < NBLK)
            def _():
                pltpu.make_async_copy(
                    acc.at[pl.ds(0, BLK)],
                    out_hbm.at[c, pl.ds(0, BLK)], wsem).wait()

    writeout(psum_hbm)

    if with_deg:
        plsc.subcore_barrier()
        load_dst(0, 0)
        load_dst(1, 1)
        load_dst(2, 2)

        def of(i, _):
            rows0[i // 8, pl.ds((i % 8) * 16, 16)] = one16
            return 0
        lax.fori_loop(0, CH * (D // 16), of, 0)

        def dstep(i, b6):
            wait_idx()
            pltpu.async_copy(rows0, acc.at[dsts_v.at[b6]], ssem0,
                             add=True)

            @pl.when(i >= 3)
            def _():
                pltpu.make_async_copy(
                    rows0, acc.at[dsts_v.at[(b6 + 3) % 6]], ssem0).wait()

            @pl.when(i + 3 < NCHUNK)
            def _():
                load_dst(i + 3, (b6 + 3) % 6)

        def dmacro(g, _):
            for u in range(6):
                i = 6 * g + u

                @pl.when(i < NCHUNK)
                def _():
                    dstep(i, u)
            return 0
        lax.fori_loop(0, (NCHUNK + 5) // 6, dmacro, 0)
        for k in range(3):
            pltpu.make_async_copy(
                rows0, acc.at[dsts_v.at[(NCHUNK - 3 + k) % 6]],
                ssem0).wait()
        plsc.subcore_barrier()
        writeout(deg_hbm)


def _make_agg(with_deg):
    mesh = plsc.VectorSubcoreMesh(core_axis_name="c", subcore_axis_name="s")
    out_type = [jax.ShapeDtypeStruct((NC, N, D), jnp.float32)]
    if with_deg:
        out_type.append(jax.ShapeDtypeStruct((NC, N, D), jnp.float32))
    scratch = [
        pltpu.VMEM_SHARED((N, D), jnp.float32),
        pltpu.VMEM((6, CH), jnp.int32),
        pltpu.VMEM((6, CH), jnp.int32),
        pltpu.VMEM((CH, D), jnp.float32),
        pltpu.VMEM((CH, D), jnp.float32),
        pltpu.VMEM((CH, D), jnp.float32),
        pltpu.VMEM((BLK, D), jnp.float32),
        pltpu.SemaphoreType.DMA,
        pltpu.SemaphoreType.DMA,
        pltpu.SemaphoreType.DMA,
        pltpu.SemaphoreType.DMA,
        pltpu.SemaphoreType.DMA,
        pltpu.SemaphoreType.DMA,
        pltpu.SemaphoreType.DMA,
        pltpu.SemaphoreType.DMA,
    ]
    return pl.kernel(
        functools.partial(_agg_body, with_deg),
        out_type=tuple(out_type),
        mesh=mesh,
        scratch_types=scratch,
        name="sc_agg_featdeg" if with_deg else "sc_agg_feat",
    )


_agg_featdeg = _make_agg(True)
_agg_feat = _make_agg(False)


def _blk_recip(d_ref, p_ref):
    d = d_ref[...]
    p = p_ref[...]
    deg = (d[0, :, 0] - p[0, :, 0]) + (d[1, :, 0] - p[1, :, 0])
    return 1.0 / jnp.clip(deg, 1.0, None)


def _tc1_body(x_ref, p_ref, d_ref, ws_ref, wn_ref, b_ref, out_ref, r_ref):
    recip = _blk_recip(d_ref, p_ref)
    p = p_ref[...]
    mean = (p[0] + p[1]) * recip[:, None]
    h = (jnp.dot(x_ref[...], ws_ref[...], preferred_element_type=jnp.float32)
         + jnp.dot(mean, wn_ref[...], preferred_element_type=jnp.float32)
         + b_ref[...])
    out_ref[...] = jnp.maximum(h, 0.0)
    r_ref[...] = jnp.broadcast_to(recip[:, None], (NB, 8))


def _tc2_body(h_ref, q_ref, r_ref, ws_ref, wn_ref, b_ref,
              wm1_ref, bm1_ref, wm2_ref, bm2_ref, out_ref):
    recip = r_ref[...][:, 0]
    q = q_ref[...]
    mean = (q[0] + q[1]) * recip[:, None]
    h = (jnp.dot(h_ref[...], ws_ref[...], preferred_element_type=jnp.float32)
         + jnp.dot(mean, wn_ref[...], preferred_element_type=jnp.float32)
         + b_ref[...])
    h = jnp.maximum(h, 0.0)
    nrm = jnp.sqrt(jnp.sum(h * h, axis=1, keepdims=True))
    h = h / jnp.maximum(nrm, 1e-12)
    t = jnp.maximum(
        jnp.dot(h, wm1_ref[...], preferred_element_type=jnp.float32)
        + bm1_ref[...], 0.0)
    logits = (jnp.dot(t, wm2_ref[...], preferred_element_type=jnp.float32)
              + bm2_ref[...])
    m = jnp.max(logits, axis=1, keepdims=True)
    e = jnp.exp(logits - m)
    out_ref[...] = e / jnp.sum(e, axis=1, keepdims=True)


NB = 1000


def _tc1(x, p, d, ws, wn, b):
    grid = (N // NB,)
    return pl.pallas_call(
        _tc1_body,
        grid=grid,
        in_specs=[
            pl.BlockSpec((NB, D), lambda i: (i, 0)),
            pl.BlockSpec((NC, NB, D), lambda i: (0, i, 0)),
            pl.BlockSpec((NC, NB, D), lambda i: (0, i, 0)),
            pl.BlockSpec((D, D), lambda i: (0, 0)),
            pl.BlockSpec((D, D), lambda i: (0, 0)),
            pl.BlockSpec((1, D), lambda i: (0, 0)),
        ],
        out_specs=[
            pl.BlockSpec((NB, D), lambda i: (i, 0)),
            pl.BlockSpec((NB, 8), lambda i: (i, 0)),
        ],
        out_shape=[
            jax.ShapeDtypeStruct((N, D), jnp.float32),
            jax.ShapeDtypeStruct((N, 8), jnp.float32),
        ],
    )(x, p, d, ws, wn, b)


def _tc2(h, q, r, ws, wn, b, wm1, bm1, wm2, bm2):
    grid = (N // NB,)
    return pl.pallas_call(
        _tc2_body,
        grid=grid,
        in_specs=[
            pl.BlockSpec((NB, D), lambda i: (i, 0)),
            pl.BlockSpec((NC, NB, D), lambda i: (0, i, 0)),
            pl.BlockSpec((NB, 8), lambda i: (i, 0)),
            pl.BlockSpec((D, D), lambda i: (0, 0)),
            pl.BlockSpec((D, D), lambda i: (0, 0)),
            pl.BlockSpec((1, D), lambda i: (0, 0)),
            pl.BlockSpec((D, 64), lambda i: (0, 0)),
            pl.BlockSpec((1, 64), lambda i: (0, 0)),
            pl.BlockSpec((64, 4), lambda i: (0, 0)),
            pl.BlockSpec((1, 4), lambda i: (0, 0)),
        ],
        out_specs=pl.BlockSpec((NB, 4), lambda i: (i, 0)),
        out_shape=jax.ShapeDtypeStruct((N, 4), jnp.float32),
    )(h, q, r, ws, wn, b, wm1, bm1, wm2, bm2)


@jax.jit
def kernel(inputs, edge_index, W1_self, W1_neigh, b1, W2_self, W2_neigh, b2,
           Wm1, bm1, Wm2, bm2):
    src = edge_index[0]
    dst = edge_index[1]

    psum1, deg2 = _agg_featdeg(inputs, src, dst)
    h1, recip = _tc1(inputs, psum1, deg2, W1_self, W1_neigh,
                     b1.reshape(1, D))
    (psum2,) = _agg_feat(h1, src, dst)
    out = _tc2(h1, psum2, recip,
               W2_self, W2_neigh, b2.reshape(1, D),
               Wm1, bm1.reshape(1, 64), Wm2, bm2.reshape(1, 4))
    return out
